# baseline (speedup 1.0000x reference)
import functools

import jax
import jax.numpy as jnp
from jax import lax
from jax.experimental import pallas as pl
from jax.experimental.pallas import tpu as pltpu
from jax.experimental.pallas import tpu_sc as plsc

_LANES = 16


def _make_kernel(B, F, V, D, NW):
    BF = B * F
    assert BF % NW == 0
    npw = BF // NW
    assert npw % F == 0
    C = 1024
    assert npw % C == 0
    nchunk = npw // C

    mesh = plsc.VectorSubcoreMesh(core_axis_name="c", subcore_axis_name="s")

    @functools.partial(
        pl.kernel,
        mesh=mesh,
        compiler_params=pltpu.CompilerParams(use_tc_tiling_on_sc=False),
        out_type=jax.ShapeDtypeStruct((BF, D), jnp.float32),
        scratch_types=[
            pltpu.VMEM((npw,), jnp.int32),
            pltpu.VMEM((npw,), jnp.int32),
            pltpu.VMEM((2, C, D), jnp.float32),
            pltpu.SemaphoreType.DMA,
            pltpu.SemaphoreType.DMA,
        ],
    )
    def emb(x_hbm, foffs_hbm, table_hbm, out_hbm, idx_v, foffs_v, rows_v, sem0, sem1):
        wid = lax.axis_index("s") * 2 + lax.axis_index("c")
        base = wid * npw

        pltpu.sync_copy(x_hbm.at[pl.ds(base, npw)], idx_v)
        pltpu.sync_copy(foffs_hbm, foffs_v)

        def add_body(i, carry):
            s = pl.ds(i * _LANES, _LANES)
            idx_v[s] = idx_v[s] + foffs_v[s]
            return carry

        lax.fori_loop(0, npw // _LANES, add_body, 0)

        sems = (sem0, sem1)

        def gather(c):
            return pltpu.async_copy(
                table_hbm.at[idx_v.at[pl.ds(c * C, C)]],
                rows_v.at[c % 2],
                sems[c % 2],
            )

        pending = gather(0)
        for c in range(nchunk):
            nxt = gather(c + 1) if c + 1 < nchunk else None
            pending.wait()
            pltpu.sync_copy(rows_v.at[c % 2], out_hbm.at[pl.ds(base + c * C, C)])
            pending = nxt

    return emb


def kernel(x, W):
    B, F = x.shape
    _, V, D = W.shape
    info = plsc.get_sparse_core_info()
    NW = info.num_cores * info.num_subcores
    x_flat = x.reshape(BF := B * F)
    W_flat = W.reshape(F * V, D)
    npw = BF // NW
    foffs = jnp.tile(jnp.arange(F, dtype=jnp.int32) * V, npw // F)
    out = _make_kernel(B, F, V, D, NW)(x_flat, foffs, W_flat)
    return out.reshape(B, F * D)
